# baseline (speedup 1.0000x reference)
import jax
import jax.numpy as jnp
from jax import lax
from jax.experimental import pallas as pl
from jax.experimental.pallas import tpu as pltpu
from jax.experimental.pallas import tpu_sc as plsc

M = 4096
D = 256
L = 16
NC = 2
NS = 16
NW = NC * NS
R = 512
RPW = R // NW
B = 512
NB = (M - R) // B


def _sc_update(x_hbm, emb_hbm, adj_hbm, pi_hbm, pf_hbm, out_hbm,
               x_v, pi_v, pf_v, wrow_v, buf, sem_g, sem_i):
    wid = lax.axis_index("s") * NC + lax.axis_index("c")
    base = wid * RPW
    pltpu.sync_copy(pi_hbm, pi_v)
    gat = pltpu.async_copy(adj_hbm.at[pi_v.at[pl.ds(0, 1)]], wrow_v, sem_g)
    emb_in = pltpu.async_copy(emb_hbm.at[pl.ds(base, RPW)], buf, sem_i)
    pltpu.sync_copy(pf_hbm, pf_v)
    pltpu.sync_copy(x_hbm, x_v)
    idxv = pi_v[pl.ds(0, L)]
    lrv = pf_v[pl.ds(0, L)]
    gat.wait()
    w16 = wrow_v[0, pl.ds(base, L)]
    rows = base + lax.broadcasted_iota(jnp.int32, (L,), 0)
    c16 = lrv * jnp.where(rows == idxv, jnp.float32(1.0),
                          jnp.where(w16 > jnp.float32(0.0), w16,
                                    jnp.float32(0.0)))
    xs = [x_v[pl.ds(k * L, L)] for k in range(D // L)]
    emb_in.wait()
    for t in range(RPW):
        cb = jnp.full((L,), c16[t], jnp.float32)
        for k in range(D // L):
            e = buf[t, pl.ds(k * L, L)]
            buf[t, pl.ds(k * L, L)] = e + cb * (xs[k] - e)
    pltpu.sync_copy(buf, out_hbm.at[pl.ds(base, RPW)])


def _tc_update(emb_ref, wcol_ref, x_ref, lr_ref, idx_ref, o_ref):
    i = pl.program_id(0)
    e = emb_ref[...]
    w = wcol_ref[...]
    xv = x_ref[...]
    lr = lr_ref[0, 0]
    idxi = idx_ref[0, 0]
    rows = R + i * B + lax.broadcasted_iota(jnp.int32, (B, 1), 0)
    c = lr * jnp.where(rows == idxi, jnp.float32(1.0),
                       jnp.where(w > jnp.float32(0.0), w, jnp.float32(0.0)))
    o_ref[...] = e + c * (xv - e)


def kernel(x, embedding_to_map, embedding_to_map_adj, iter, idx, max_iter):
    lr = jnp.float32(0.1) * (jnp.float32(1.0)
                             - jnp.float32(iter) / jnp.float32(max_iter))
    idx32 = jnp.asarray(idx, jnp.int32)
    p_idx = jnp.full((L,), idx32, jnp.int32)
    p_lr = jnp.full((L,), lr, jnp.float32)

    mesh = plsc.VectorSubcoreMesh(core_axis_name="c", subcore_axis_name="s")
    sc_som = pl.kernel(
        _sc_update,
        out_type=jax.ShapeDtypeStruct((R, D), jnp.float32),
        mesh=mesh,
        scratch_types=[
            pltpu.VMEM((D,), jnp.float32),
            pltpu.VMEM((L,), jnp.int32),
            pltpu.VMEM((L,), jnp.float32),
            pltpu.VMEM((1, M), jnp.float32),
            pltpu.VMEM((RPW, D), jnp.float32),
            pltpu.SemaphoreType.DMA,
            pltpu.SemaphoreType.DMA,
        ],
    )
    sc_out = sc_som(x, embedding_to_map, embedding_to_map_adj, p_idx, p_lr)

    w_col = jnp.reshape(
        lax.dynamic_slice(embedding_to_map_adj, (idx32, jnp.int32(0)), (1, M)),
        (M, 1))
    x2 = jnp.reshape(x, (1, D))
    lr_arr = jnp.full((1, 1), lr, jnp.float32)
    idx_arr = jnp.full((1, 1), idx32, jnp.int32)
    tc_out = pl.pallas_call(
        _tc_update,
        grid=(NB,),
        in_specs=[
            pl.BlockSpec((B, D), lambda i: (R // B + i, 0)),
            pl.BlockSpec((B, 1), lambda i: (R // B + i, 0)),
            pl.BlockSpec((1, D), lambda i: (0, 0)),
            pl.BlockSpec((1, 1), lambda i: (0, 0)),
            pl.BlockSpec((1, 1), lambda i: (0, 0)),
        ],
        out_specs=pl.BlockSpec((B, D), lambda i: (R // B + i, 0)),
        out_shape=jax.ShapeDtypeStruct((M, D), jnp.float32),
    )(embedding_to_map, w_col, x2, lr_arr, idx_arr)

    return lax.dynamic_update_slice(tc_out, sc_out, (0, 0))
